# baseline (speedup 1.0000x reference)
import functools

import jax
import jax.numpy as jnp
from jax import lax
from jax.experimental import pallas as pl
from jax.experimental.pallas import tpu as pltpu
from jax.experimental.pallas import tpu_sc as plsc

HIDDEN_DIM = 512
BATCH = 16384
LANES = 16

_info = plsc.get_sparse_core_info()
NC, NS = _info.num_cores, _info.num_subcores
NW = NC * NS
B_PER_W = BATCH // NW

QROWS = 4
NQPAT = 2 ** QROWS
NGRP = B_PER_W // LANES
N_Q = B_PER_W // QROWS

NCOL = 4
CW = HIDDEN_DIM // NCOL
JH = CW // LANES

_DB_BITS = [0, 0, 0, 0, 1, 0, 0, 1, 1, 0, 1, 0, 1, 1, 1, 1, 0, 0, 0]
_DB_OFF = [0, 1, 2, 5, 3, 9, 6, 11, 15, 4, 8, 10, 14, 7, 13, 12]


def _perm(x, idx):
    return lax.gather(
        x, idx.reshape(LANES, 1),
        lax.GatherDimensionNumbers(
            offset_dims=(), collapsed_slice_dims=(0,), start_index_map=(0,)),
        (1,), mode=lax.GatherScatterMode.PROMISE_IN_BOUNDS)


def _mesh_kernel():
    mesh = plsc.VectorSubcoreMesh(core_axis_name="c", subcore_axis_name="s")

    @functools.partial(
        pl.kernel,
        mesh=mesh,
        out_type=jax.ShapeDtypeStruct((BATCH * HIDDEN_DIM,), jnp.float32),
        scratch_types=[
            pltpu.VMEM((B_PER_W,), jnp.int32),
            pltpu.VMEM((2, HIDDEN_DIM), jnp.float32),
            pltpu.VMEM((19 * HIDDEN_DIM,), jnp.float32),
            pltpu.VMEM((B_PER_W,), jnp.int32),
            pltpu.SemaphoreType.DMA,
            pltpu.SemaphoreType.DMA,
            pltpu.SemaphoreType.DMA,
        ],
    )
    def body(table_hbm, idx_hbm, out_hbm, idx_v, tab_v, quads, pat_v,
             sem, semi, semt):
        wid = lax.axis_index("s") * NC + lax.axis_index("c")
        base = wid * B_PER_W
        cp_idx = pltpu.async_copy(idx_hbm.at[wid], idx_v, semi)
        cp_tab = pltpu.async_copy(table_hbm, tab_v, semt)
        cp_tab.wait()

        lane = lax.iota(jnp.int32, LANES)
        wvec = 8 >> jnp.bitwise_and(lane, 3)

        for h in range(NCOL):
            c0 = h * CW
            w0 = [tab_v[0, pl.ds(c0 + j * LANES, LANES)] for j in range(JH)]
            w1 = [tab_v[1, pl.ds(c0 + j * LANES, LANES)] for j in range(JH)]
            for rr, bit in enumerate(_DB_BITS):
                src = w1 if bit else w0
                for j in range(JH):
                    quads[pl.ds(rr * HIDDEN_DIM + c0 + j * LANES,
                                LANES)] = src[j]

        off_vec = lane * 0
        for p in range(1, NQPAT):
            off_vec = jnp.where(lane == p, _DB_OFF[p], off_vec)

        cp_idx.wait()
        for t in range(NGRP):
            v = idx_v[pl.ds(t * LANES, LANES)]
            s = v * wvec
            s = s + _perm(s, jnp.bitwise_xor(lane, 1))
            s = s + _perm(s, jnp.bitwise_xor(lane, 2))
            pat_v[pl.ds(t * LANES, LANES)] = _perm(off_vec, s) * HIDDEN_DIM

        QW = QROWS * HIDDEN_DIM

        def issue_body(t, _):
            pv = pat_v[pl.ds(t * LANES, LANES)]
            row0 = base + t * LANES
            for i in range(LANES // QROWS):
                pltpu.async_copy(
                    quads.at[pl.ds(pl.multiple_of(pv[QROWS * i], 512), QW)],
                    out_hbm.at[pl.ds(pl.multiple_of(
                        (row0 + i * QROWS) * HIDDEN_DIM, 512), QW)],
                    sem)
            return 0

        lax.fori_loop(0, NGRP, issue_body, 0)

        def drain_body(t, _):
            pltpu.make_async_copy(
                quads.at[pl.ds(0, QROWS * HIDDEN_DIM)],
                out_hbm.at[pl.ds(base * HIDDEN_DIM, QROWS * HIDDEN_DIM)],
                sem).wait()
            return 0

        lax.fori_loop(0, N_Q, drain_body, 0)

    return body


_sc_lookup = _mesh_kernel()


@jax.jit
def kernel(domain_ids, embed_weight):
    ids = domain_ids.astype(jnp.int32).reshape(NW, B_PER_W)
    return _sc_lookup(embed_weight, ids).reshape(BATCH, HIDDEN_DIM)


# speedup vs baseline: 1.7431x; 1.7431x over previous
import functools

import jax
import jax.numpy as jnp
from jax import lax
from jax.experimental import pallas as pl
from jax.experimental.pallas import tpu as pltpu
from jax.experimental.pallas import tpu_sc as plsc

HIDDEN_DIM = 512
BATCH = 16384
LANES = 16

_info = plsc.get_sparse_core_info()
NC, NS = _info.num_cores, _info.num_subcores
NW = NC * NS
B_PER_W = BATCH // NW

QROWS = 4
NQPAT = 2 ** QROWS
NGRP = B_PER_W // LANES
HEAD_ROWS = 64
HEAD_GRPS = HEAD_ROWS // LANES
Q_GRP0 = HEAD_ROWS // LANES
N_Q = (B_PER_W - HEAD_ROWS) // QROWS

NCOL = 4
CW = HIDDEN_DIM // NCOL
JH = CW // LANES


def _perm(x, idx):
    return lax.gather(
        x, idx.reshape(LANES, 1),
        lax.GatherDimensionNumbers(
            offset_dims=(), collapsed_slice_dims=(0,), start_index_map=(0,)),
        (1,), mode=lax.GatherScatterMode.PROMISE_IN_BOUNDS)


def _mesh_kernel():
    mesh = plsc.VectorSubcoreMesh(core_axis_name="c", subcore_axis_name="s")

    @functools.partial(
        pl.kernel,
        mesh=mesh,
        out_type=jax.ShapeDtypeStruct((BATCH, HIDDEN_DIM), jnp.float32),
        scratch_types=[
            pltpu.VMEM((B_PER_W,), jnp.int32),
            pltpu.VMEM((2, HIDDEN_DIM), jnp.float32),
            pltpu.VMEM((NQPAT, QROWS, HIDDEN_DIM), jnp.float32),
            pltpu.VMEM((B_PER_W,), jnp.int32),
            pltpu.SemaphoreType.DMA,
            pltpu.SemaphoreType.DMA,
            pltpu.SemaphoreType.DMA,
            pltpu.SemaphoreType.DMA,
        ],
    )
    def body(table_hbm, idx_hbm, out_hbm, idx_v, tab_v, quads, pat_v,
             sem, semr, semi, semt):
        wid = lax.axis_index("s") * NC + lax.axis_index("c")
        base = wid * B_PER_W
        cp_idx = pltpu.async_copy(idx_hbm.at[wid], idx_v, semi)
        cp_tab = pltpu.async_copy(table_hbm, tab_v, semt)
        cp_tab.wait()
        cp_idx.wait()

        def head_body(t, _):
            v = idx_v[pl.ds(t * LANES, LANES)]
            row0 = base + t * LANES
            for r in range(LANES):
                pltpu.async_copy(tab_v.at[v[r]], out_hbm.at[row0 + r], semr)
            return 0

        lax.fori_loop(0, HEAD_GRPS, head_body, 0)

        lane = lax.iota(jnp.int32, LANES)
        wvec = 8 >> jnp.bitwise_and(lane, 3)

        for h in range(NCOL):
            c0 = h * CW
            w0 = [tab_v[0, pl.ds(c0 + j * LANES, LANES)] for j in range(JH)]
            w1 = [tab_v[1, pl.ds(c0 + j * LANES, LANES)] for j in range(JH)]
            for q in range(NQPAT):
                for rr in range(QROWS):
                    src = w1 if (q >> (QROWS - 1 - rr)) & 1 else w0
                    for j in range(JH):
                        quads[q, rr, pl.ds(c0 + j * LANES, LANES)] = src[j]

        for t in range(Q_GRP0, NGRP):
            v = idx_v[pl.ds(t * LANES, LANES)]
            s = v * wvec
            s = s + _perm(s, jnp.bitwise_xor(lane, 1))
            s = s + _perm(s, jnp.bitwise_xor(lane, 2))
            pat_v[pl.ds(t * LANES, LANES)] = s

        def issue_body(t, _):
            pv = pat_v[pl.ds(t * LANES, LANES)]
            row0 = base + t * LANES
            for i in range(LANES // QROWS):
                pltpu.async_copy(
                    quads.at[pv[QROWS * i]],
                    out_hbm.at[pl.ds(row0 + i * QROWS, QROWS)], sem)
            return 0

        lax.fori_loop(Q_GRP0, NGRP, issue_body, 0)

        def drain_body(t, _):
            pltpu.make_async_copy(
                quads.at[0], out_hbm.at[pl.ds(base, QROWS)], sem).wait()
            return 0

        lax.fori_loop(0, N_Q, drain_body, 0)

        def drain_head(t, _):
            pltpu.make_async_copy(
                tab_v.at[0], out_hbm.at[base], semr).wait()
            return 0

        lax.fori_loop(0, HEAD_ROWS, drain_head, 0)

    return body


_sc_lookup = _mesh_kernel()


@jax.jit
def kernel(domain_ids, embed_weight):
    ids = domain_ids.astype(jnp.int32).reshape(NW, B_PER_W)
    return _sc_lookup(embed_weight, ids)


# speedup vs baseline: 1.8378x; 1.0543x over previous
import functools

import jax
import jax.numpy as jnp
from jax import lax
from jax.experimental import pallas as pl
from jax.experimental.pallas import tpu as pltpu
from jax.experimental.pallas import tpu_sc as plsc

HIDDEN_DIM = 512
BATCH = 16384
LANES = 16

_info = plsc.get_sparse_core_info()
NC, NS = _info.num_cores, _info.num_subcores
NW = NC * NS
B_PER_W = BATCH // NW

QROWS = 4
NQPAT = 2 ** QROWS
NGRP = B_PER_W // LANES
N_Q = B_PER_W // QROWS

NCOL = 4
CW = HIDDEN_DIM // NCOL
JH = CW // LANES


def _perm(x, idx):
    return lax.gather(
        x, idx.reshape(LANES, 1),
        lax.GatherDimensionNumbers(
            offset_dims=(), collapsed_slice_dims=(0,), start_index_map=(0,)),
        (1,), mode=lax.GatherScatterMode.PROMISE_IN_BOUNDS)


def _mesh_kernel():
    mesh = plsc.VectorSubcoreMesh(core_axis_name="c", subcore_axis_name="s")

    @functools.partial(
        pl.kernel,
        mesh=mesh,
        out_type=jax.ShapeDtypeStruct((BATCH, HIDDEN_DIM), jnp.float32),
        scratch_types=[
            pltpu.VMEM((B_PER_W,), jnp.int32),
            pltpu.VMEM((2, HIDDEN_DIM), jnp.float32),
            pltpu.VMEM((NQPAT, QROWS, HIDDEN_DIM), jnp.float32),
            pltpu.VMEM((B_PER_W,), jnp.int32),
            pltpu.SemaphoreType.DMA,
            pltpu.SemaphoreType.DMA,
            pltpu.SemaphoreType.DMA,
        ],
    )
    def body(table_hbm, idx_hbm, out_hbm, idx_v, tab_v, quads, pat_v,
             sem, semi, semt):
        wid = lax.axis_index("s") * NC + lax.axis_index("c")
        base = wid * B_PER_W
        cp_idx = pltpu.async_copy(idx_hbm.at[wid], idx_v, semi)
        cp_tab = pltpu.async_copy(table_hbm, tab_v, semt)

        lane = lax.iota(jnp.int32, LANES)
        wvec = 8 >> jnp.bitwise_and(lane, 3)

        cp_idx.wait()
        for t in range(NGRP):
            v = idx_v[pl.ds(t * LANES, LANES)]
            s = v * wvec
            s = s + _perm(s, jnp.bitwise_xor(lane, 1))
            s = s + _perm(s, jnp.bitwise_xor(lane, 2))
            pat_v[pl.ds(t * LANES, LANES)] = s

        cp_tab.wait()
        for h in range(NCOL):
            c0 = h * CW
            w0 = [tab_v[0, pl.ds(c0 + j * LANES, LANES)] for j in range(JH)]
            w1 = [tab_v[1, pl.ds(c0 + j * LANES, LANES)] for j in range(JH)]
            for q in range(NQPAT):
                for rr in range(QROWS):
                    src = w1 if (q >> (QROWS - 1 - rr)) & 1 else w0
                    for j in range(JH):
                        quads[q, rr, pl.ds(c0 + j * LANES, LANES)] = src[j]

        def issue_body(t, _):
            pv = pat_v[pl.ds(t * LANES, LANES)]
            row0 = base + t * LANES
            for i in range(LANES // QROWS):
                pltpu.async_copy(
                    quads.at[pv[QROWS * i]],
                    out_hbm.at[pl.ds(row0 + i * QROWS, QROWS)], sem)
            return 0

        lax.fori_loop(0, NGRP, issue_body, 0)

        def drain_body(t, _):
            pltpu.make_async_copy(
                quads.at[0], out_hbm.at[pl.ds(base, QROWS)], sem).wait()
            return 0

        lax.fori_loop(0, N_Q, drain_body, 0)

    return body


_sc_lookup = _mesh_kernel()


@jax.jit
def kernel(domain_ids, embed_weight):
    ids = domain_ids.astype(jnp.int32).reshape(NW, B_PER_W)
    return _sc_lookup(embed_weight, ids)


# speedup vs baseline: 2.0749x; 1.1290x over previous
import functools

import jax
import jax.numpy as jnp
from jax import lax
from jax.experimental import pallas as pl
from jax.experimental.pallas import tpu as pltpu
from jax.experimental.pallas import tpu_sc as plsc

HIDDEN_DIM = 512
BATCH = 16384
LANES = 16

_info = plsc.get_sparse_core_info()
NC, NS = _info.num_cores, _info.num_subcores
NW = NC * NS
B_PER_W = BATCH // NW

QROWS = 2
NQPAT = 2 ** QROWS
NGRP = B_PER_W // LANES
N_Q = B_PER_W // QROWS

NCOL = 4
CW = HIDDEN_DIM // NCOL
JH = CW // LANES


def _perm(x, idx):
    return lax.gather(
        x, idx.reshape(LANES, 1),
        lax.GatherDimensionNumbers(
            offset_dims=(), collapsed_slice_dims=(0,), start_index_map=(0,)),
        (1,), mode=lax.GatherScatterMode.PROMISE_IN_BOUNDS)


def _mesh_kernel():
    mesh = plsc.VectorSubcoreMesh(core_axis_name="c", subcore_axis_name="s")

    @functools.partial(
        pl.kernel,
        mesh=mesh,
        out_type=jax.ShapeDtypeStruct((BATCH, HIDDEN_DIM), jnp.float32),
        scratch_types=[
            pltpu.VMEM((B_PER_W,), jnp.int32),
            pltpu.VMEM((2, HIDDEN_DIM), jnp.float32),
            pltpu.VMEM((NQPAT, QROWS, HIDDEN_DIM), jnp.float32),
            pltpu.VMEM((B_PER_W,), jnp.int32),
            pltpu.SemaphoreType.DMA,
            pltpu.SemaphoreType.DMA,
            pltpu.SemaphoreType.DMA,
        ],
    )
    def body(table_hbm, idx_hbm, out_hbm, idx_v, tab_v, quads, pat_v,
             sem, semi, semt):
        wid = lax.axis_index("s") * NC + lax.axis_index("c")
        base = wid * B_PER_W
        cp_idx = pltpu.async_copy(idx_hbm.at[wid], idx_v, semi)
        cp_tab = pltpu.async_copy(table_hbm, tab_v, semt)

        lane = lax.iota(jnp.int32, LANES)
        wvec = 2 >> jnp.bitwise_and(lane, 1)

        cp_idx.wait()
        for t in range(NGRP):
            v = idx_v[pl.ds(t * LANES, LANES)]
            s = v * wvec
            s = s + _perm(s, jnp.bitwise_xor(lane, 1))
            pat_v[pl.ds(t * LANES, LANES)] = s

        cp_tab.wait()
        for h in range(NCOL):
            c0 = h * CW
            w0 = [tab_v[0, pl.ds(c0 + j * LANES, LANES)] for j in range(JH)]
            w1 = [tab_v[1, pl.ds(c0 + j * LANES, LANES)] for j in range(JH)]
            for q in range(NQPAT):
                for rr in range(QROWS):
                    src = w1 if (q >> (QROWS - 1 - rr)) & 1 else w0
                    for j in range(JH):
                        quads[q, rr, pl.ds(c0 + j * LANES, LANES)] = src[j]

        def issue_body(t, _):
            pv = pat_v[pl.ds(t * LANES, LANES)]
            row0 = base + t * LANES
            for i in range(LANES // QROWS):
                pltpu.async_copy(
                    quads.at[pv[QROWS * i]],
                    out_hbm.at[pl.ds(row0 + i * QROWS, QROWS)], sem)
            return 0

        lax.fori_loop(0, NGRP, issue_body, 0)

        def drain_body(t, _):
            pltpu.make_async_copy(
                quads.at[0], out_hbm.at[pl.ds(base, QROWS)], sem).wait()
            return 0

        lax.fori_loop(0, N_Q, drain_body, 0)

    return body


_sc_lookup = _mesh_kernel()


@jax.jit
def kernel(domain_ids, embed_weight):
    ids = domain_ids.astype(jnp.int32).reshape(NW, B_PER_W)
    return _sc_lookup(embed_weight, ids)


# speedup vs baseline: 2.0766x; 1.0009x over previous
import functools

import jax
import jax.numpy as jnp
from jax import lax
from jax.experimental import pallas as pl
from jax.experimental.pallas import tpu as pltpu
from jax.experimental.pallas import tpu_sc as plsc

HIDDEN_DIM = 512
BATCH = 16384
LANES = 16

_info = plsc.get_sparse_core_info()
NC, NS = _info.num_cores, _info.num_subcores
NW = NC * NS
B_PER_W = BATCH // NW

QROWS = 2
NQPAT = 2 ** QROWS
NGRP = B_PER_W // LANES
N_Q = B_PER_W // QROWS

NCOL = 4
CW = HIDDEN_DIM // NCOL
JH = CW // LANES


def _perm(x, idx):
    return lax.gather(
        x, idx.reshape(LANES, 1),
        lax.GatherDimensionNumbers(
            offset_dims=(), collapsed_slice_dims=(0,), start_index_map=(0,)),
        (1,), mode=lax.GatherScatterMode.PROMISE_IN_BOUNDS)


def _mesh_kernel():
    mesh = plsc.VectorSubcoreMesh(core_axis_name="c", subcore_axis_name="s")

    @functools.partial(
        pl.kernel,
        mesh=mesh,
        out_type=jax.ShapeDtypeStruct((BATCH, HIDDEN_DIM), jnp.float32),
        scratch_types=[
            pltpu.VMEM((B_PER_W,), jnp.int32),
            pltpu.VMEM((2, HIDDEN_DIM), jnp.float32),
            pltpu.VMEM((NQPAT, QROWS, HIDDEN_DIM), jnp.float32),
            pltpu.SemaphoreType.DMA,
            pltpu.SemaphoreType.DMA,
            pltpu.SemaphoreType.DMA,
        ],
    )
    def body(table_hbm, idx_hbm, out_hbm, idx_v, tab_v, quads,
             sem, semi, semt):
        wid = lax.axis_index("s") * NC + lax.axis_index("c")
        base = wid * B_PER_W
        cp_idx = pltpu.async_copy(idx_hbm.at[wid], idx_v, semi)
        cp_tab = pltpu.async_copy(table_hbm, tab_v, semt)

        lane = lax.iota(jnp.int32, LANES)
        wvec = 2 >> jnp.bitwise_and(lane, 1)

        cp_tab.wait()
        for h in range(NCOL):
            c0 = h * CW
            w0 = [tab_v[0, pl.ds(c0 + j * LANES, LANES)] for j in range(JH)]
            w1 = [tab_v[1, pl.ds(c0 + j * LANES, LANES)] for j in range(JH)]
            for q in range(NQPAT):
                for rr in range(QROWS):
                    src = w1 if (q >> (QROWS - 1 - rr)) & 1 else w0
                    for j in range(JH):
                        quads[q, rr, pl.ds(c0 + j * LANES, LANES)] = src[j]

        cp_idx.wait()

        def issue_body(t, _):
            v = idx_v[pl.ds(t * LANES, LANES)]
            s = v * wvec
            s = s + _perm(s, jnp.bitwise_xor(lane, 1))
            row0 = base + t * LANES
            for i in range(LANES // QROWS):
                pltpu.async_copy(
                    quads.at[s[QROWS * i]],
                    out_hbm.at[pl.ds(row0 + i * QROWS, QROWS)], sem)
            return 0

        lax.fori_loop(0, NGRP, issue_body, 0)

        def drain_body(t, _):
            pltpu.make_async_copy(
                quads.at[0], out_hbm.at[pl.ds(base, QROWS)], sem).wait()
            return 0

        lax.fori_loop(0, N_Q, drain_body, 0)

    return body


_sc_lookup = _mesh_kernel()


@jax.jit
def kernel(domain_ids, embed_weight):
    ids = domain_ids.astype(jnp.int32).reshape(NW, B_PER_W)
    return _sc_lookup(embed_weight, ids)
